# baseline (speedup 1.0000x reference)
import jax
import jax.numpy as jnp
import numpy as np
from jax import lax
from jax.experimental import pallas as pl
from jax.experimental.pallas import tpu as pltpu

K_CL = 5
EPS = 0.001


def _body(bs_ref, table_ref, w1_ref, b1_ref, w2_ref, b2_ref,
          soft_ref, rank_ref, scores_ref):
    B, rows, col = table_ref.shape
    x2 = table_ref[...].reshape(B * rows, col)
    h1 = jnp.maximum(
        lax.dot_general(x2, w1_ref[...], (((1,), (0,)), ((), ())),
                        preferred_element_type=jnp.float32) + b1_ref[...], 0.0)
    h = jnp.maximum(
        lax.dot_general(h1, w2_ref[...], (((1,), (0,)), ((), ())),
                        preferred_element_type=jnp.float32) + b2_ref[...], 0.0)

    x0 = h[:rows]
    init_idx = np.linspace(0, rows - 1, K_CL).astype(np.int32)
    c0 = jnp.concatenate([x0[int(i):int(i) + 1, :] for i in init_idx], axis=0)
    ones_col = jnp.ones((rows, 1), dtype=jnp.float32)
    kvec = lax.broadcasted_iota(jnp.int32, (1, K_CL), 1)

    def km_body(_, carry):
        c, _lab = carry
        best = jnp.sum((x0 - c[0:1, :]) ** 2, axis=1, keepdims=True)
        lab = jnp.zeros((rows, 1), dtype=jnp.int32)
        for k in range(1, K_CL):
            dk = jnp.sum((x0 - c[k:k + 1, :]) ** 2, axis=1, keepdims=True)
            better = dk < best
            lab = jnp.where(better, k, lab)
            best = jnp.where(better, dk, best)
        onehot = (lab == kvec).astype(jnp.float32)
        counts = lax.dot_general(onehot, ones_col, (((0,), (0,)), ((), ())),
                                 preferred_element_type=jnp.float32)
        csum = lax.dot_general(onehot, x0, (((0,), (0,)), ((), ())),
                               preferred_element_type=jnp.float32)
        return csum / jnp.maximum(counts, 1.0), lab

    _, labels = lax.fori_loop(0, 10, km_body,
                              (c0, jnp.zeros((rows, 1), jnp.int32)))

    onehot = (labels == kvec).astype(jnp.float32)
    counts = lax.dot_general(onehot, ones_col, (((0,), (0,)), ((), ())),
                             preferred_element_type=jnp.float32)
    inv_counts = 1.0 / jnp.maximum(counts, 1.0)
    lab_f = labels.astype(jnp.float32)

    bs_f = bs_ref[0].astype(jnp.float32)
    CH = 512
    nch = rows // CH
    jio = lax.broadcasted_iota(jnp.int32, (CH, rows), 1)
    iio0 = lax.broadcasted_iota(jnp.int32, (CH, rows), 0)

    for b in range(B):
        hb = h[b * rows:(b + 1) * rows]
        centers = lax.dot_general(onehot, hb, (((0,), (0,)), ((), ())),
                                  preferred_element_type=jnp.float32) * inv_counts
        cdata = lax.dot_general(onehot, centers, (((1,), (0,)), ((), ())),
                                preferred_element_type=jnp.float32,
                                precision=lax.Precision.HIGHEST)
        dist = jnp.mean((hb - cdata) ** 2, axis=1, keepdims=True)
        mn = jnp.min(dist)
        mx = jnp.max(dist)
        sco = (dist - mn) / (mx - mn) + lab_f
        mn2 = jnp.min(sco)
        mx2 = jnp.max(sco)
        sca = (sco - mn2) / (mx2 - mn2) * float(B)
        scores_ref[b] = sco

        s_row = lax.transpose(sco, (1, 0))
        t_row5 = lax.transpose(sca, (1, 0)) * (0.5 / EPS)
        for ci in range(nch):
            lo = ci * CH
            s_blk = sco[lo:lo + CH]
            t_blk5 = sca[lo:lo + CH] * (0.5 / EPS)

            sig = jnp.tanh(t_blk5 - t_row5) * 0.5 + 0.5
            soft = lax.dot_general(sig, ones_col, (((1,), (0,)), ((), ())),
                                   preferred_element_type=jnp.float32)
            soft_ref[b, lo:lo + CH, :] = soft + 0.5

            tri = (jio < iio0 + lo).astype(jnp.float32)
            lt = s_row < s_blk
            eq = s_row == s_blk
            cnt = jnp.where(eq, tri, lt.astype(jnp.float32))
            hard_f = lax.dot_general(cnt, ones_col, (((1,), (0,)), ((), ())),
                                     preferred_element_type=jnp.float32)
            rank_ref[b, lo:lo + CH, :] = (jnp.floor(hard_f / bs_f) + 1.0
                                          ).astype(jnp.int32)


def kernel(table, W1, b1, W2, b2, BlockSize, current_epoch):
    B, rows, col = table.shape
    bs = jnp.asarray(BlockSize, jnp.int32).reshape(1)
    soft, rank, scores = pl.pallas_call(
        _body,
        in_specs=[
            pl.BlockSpec(memory_space=pltpu.SMEM),
            pl.BlockSpec(memory_space=pltpu.VMEM),
            pl.BlockSpec(memory_space=pltpu.VMEM),
            pl.BlockSpec(memory_space=pltpu.VMEM),
            pl.BlockSpec(memory_space=pltpu.VMEM),
            pl.BlockSpec(memory_space=pltpu.VMEM),
        ],
        out_shape=(
            jax.ShapeDtypeStruct((B, rows, 1), jnp.float32),
            jax.ShapeDtypeStruct((B, rows, 1), jnp.int32),
            jax.ShapeDtypeStruct((B, rows, 1), jnp.float32),
        ),
    )(bs, table, W1, b1.reshape(1, -1), W2, b2.reshape(1, -1))
    return soft, rank, scores
